# baseline (speedup 1.0000x reference)
import jax
import jax.numpy as jnp
from jax.experimental import pallas as pl

BLK_S = 1024
BLK_B = 1


def _add_kernel(x_ref, e_ref, o_ref):
    o_ref[...] = x_ref[...] + e_ref[...][None, :, :]


def kernel(x, emb):
    b, s, d = x.shape
    grid = (s // BLK_S, b // BLK_B)
    return pl.pallas_call(
        _add_kernel,
        grid=grid,
        in_specs=[
            pl.BlockSpec((BLK_B, BLK_S, d), lambda i, j: (j, i, 0)),
            pl.BlockSpec((BLK_S, d), lambda i, j: (i, 0)),
        ],
        out_specs=pl.BlockSpec((BLK_B, BLK_S, d), lambda i, j: (j, i, 0)),
        out_shape=jax.ShapeDtypeStruct((b, s, d), x.dtype),
    )(x, emb)


# speedup vs baseline: 1.0396x; 1.0396x over previous
import jax
import jax.numpy as jnp
from jax.experimental import pallas as pl

BLK_S = 1024
BLK_B = 2


def _add_kernel(x_ref, e_ref, o_ref):
    o_ref[...] = x_ref[...] + e_ref[...][None, :, :]


def kernel(x, emb):
    b, s, d = x.shape
    grid = (s // BLK_S, b // BLK_B)
    return pl.pallas_call(
        _add_kernel,
        grid=grid,
        in_specs=[
            pl.BlockSpec((BLK_B, BLK_S, d), lambda i, j: (j, i, 0)),
            pl.BlockSpec((BLK_S, d), lambda i, j: (i, 0)),
        ],
        out_specs=pl.BlockSpec((BLK_B, BLK_S, d), lambda i, j: (j, i, 0)),
        out_shape=jax.ShapeDtypeStruct((b, s, d), x.dtype),
    )(x, emb)
